# Initial kernel scaffold; baseline (speedup 1.0000x reference)
import jax
import jax.numpy as jnp
from jax.experimental import pallas as pl


def kernel(input, weight):
    raise NotImplementedError("write your pallas kernel here")


# speedup vs baseline: 3.7893x; 3.7893x over previous
import functools

import jax
import jax.numpy as jnp
from jax import lax
from jax.experimental import pallas as pl
from jax.experimental.pallas import tpu as pltpu
from jax.experimental.pallas import tpu_sc as plsc

N_PARTS = 24
POSE_DIM = 10
ROW = N_PARTS * POSE_DIM

NC = 2
NS = 16
NW = NC * NS

CHUNK = 128


@functools.lru_cache(maxsize=None)
def _make_gather(B: int, V: int):
    assert B % (8 * NW) == 0
    b_per_w = B // NW
    nchunk = b_per_w // CHUNK
    assert nchunk * CHUNK == b_per_w
    mesh = plsc.VectorSubcoreMesh(core_axis_name="c", subcore_axis_name="s")

    @functools.partial(
        pl.kernel,
        mesh=mesh,
        out_type=jax.ShapeDtypeStruct((B, ROW), jnp.float32),
        compiler_params=pltpu.CompilerParams(use_tc_tiling_on_sc=False),
        scratch_types=[
            pltpu.VMEM((nchunk, CHUNK), jnp.int32),
            pltpu.VMEM((CHUNK, ROW), jnp.float32),
            pltpu.SemaphoreType.DMA,
        ],
    )
    def gather_rows(idx_hbm, table_hbm, out_hbm, idx_v, rows_v, gsem):
        wid = lax.axis_index("s") * NC + lax.axis_index("c")
        base = wid * b_per_w
        for j in range(nchunk):
            pltpu.sync_copy(idx_hbm.at[pl.ds(base + j * CHUNK, CHUNK)],
                            idx_v.at[j])
            pltpu.async_copy(table_hbm.at[idx_v.at[j]], rows_v, gsem).wait()
            pltpu.sync_copy(rows_v,
                            out_hbm.at[pl.ds(base + j * CHUNK, CHUNK)])

    return gather_rows


def kernel(input, weight):
    B = input.shape[0]
    table = weight.reshape(weight.shape[0], ROW)
    flat = _make_gather(B, table.shape[0])(input, table)
    emb = flat.reshape(B, N_PARTS, POSE_DIM)
    return emb[..., 0:4], emb[..., 4:7], emb[..., 7:10]


# speedup vs baseline: 22.3034x; 5.8859x over previous
import functools

import jax
import jax.numpy as jnp
from jax import lax
from jax.experimental import pallas as pl
from jax.experimental.pallas import tpu as pltpu
from jax.experimental.pallas import tpu_sc as plsc

N_PARTS = 24
POSE_DIM = 10
ROW = N_PARTS * POSE_DIM

NC = 2
NS = 16
NW = NC * NS

L = 16
OUT_CHUNK = 2048


@functools.lru_cache(maxsize=None)
def _make_gather(B: int, V: int):
    rows_per_w = -(-ROW // NW)
    nchunk = B // OUT_CHUNK
    assert nchunk * OUT_CHUNK == B
    mesh = plsc.VectorSubcoreMesh(core_axis_name="c", subcore_axis_name="s")

    @functools.partial(
        pl.kernel,
        mesh=mesh,
        out_type=jax.ShapeDtypeStruct((ROW, B), jnp.float32),
        compiler_params=pltpu.CompilerParams(use_tc_tiling_on_sc=True,
                                             needs_layout_passes=False),
        scratch_types=[
            pltpu.VMEM((V,), jnp.float32),
            pltpu.VMEM((B,), jnp.int32),
            pltpu.VMEM((OUT_CHUNK,), jnp.float32),
        ],
    )
    def lane_gather(table_hbm, idx_hbm, out_hbm, col_v, idx_v, out_v):
        wid = lax.axis_index("s") * NC + lax.axis_index("c")
        pltpu.sync_copy(idx_hbm, idx_v)
        for j in range(rows_per_w):
            r = j * NW + wid
            @pl.when(r < ROW)
            def _process_row():
                pltpu.sync_copy(table_hbm.at[r], col_v)
                for jc in range(nchunk):
                    def body(i, _):
                        b0 = i * (4 * L)
                        for u in range(4):
                            o = b0 + u * L
                            g = plsc.load_gather(
                                col_v,
                                [idx_v[pl.ds(jc * OUT_CHUNK + o, L)]])
                            out_v[pl.ds(o, L)] = g
                        return 0
                    lax.fori_loop(0, OUT_CHUNK // (4 * L), body, 0)
                    pltpu.sync_copy(
                        out_v, out_hbm.at[r, pl.ds(jc * OUT_CHUNK, OUT_CHUNK)])

    return lane_gather


def kernel(input, weight):
    B = input.shape[0]
    V = weight.shape[0]
    tableT = weight.transpose(2, 1, 0).reshape(ROW, V)
    out = _make_gather(B, V)(tableT, input)
    out3 = out.reshape(POSE_DIM, N_PARTS, B)
    rotation = out3[0:4].transpose(2, 1, 0)
    translation = out3[4:7].transpose(2, 1, 0)
    scale = out3[7:10].transpose(2, 1, 0)
    return rotation, translation, scale


# speedup vs baseline: 30.8757x; 1.3843x over previous
import functools

import jax
import jax.numpy as jnp
from jax import lax
from jax.experimental import pallas as pl
from jax.experimental.pallas import tpu as pltpu
from jax.experimental.pallas import tpu_sc as plsc

N_PARTS = 24
POSE_DIM = 10
ROW = N_PARTS * POSE_DIM
ROT_ROWS = 4 * N_PARTS
TRA_ROWS = 3 * N_PARTS
SCA_ROWS = 3 * N_PARTS

NC = 2
NS = 16
NW = NC * NS

L = 16
UNROLL = 8
OUT_CHUNK = 8192


@functools.lru_cache(maxsize=None)
def _make_gather(B: int, V: int):
    rows_per_w = -(-ROW // NW)
    nchunk = B // OUT_CHUNK
    assert nchunk * OUT_CHUNK == B
    mesh = plsc.VectorSubcoreMesh(core_axis_name="c", subcore_axis_name="s")

    @functools.partial(
        pl.kernel,
        mesh=mesh,
        out_type=(
            jax.ShapeDtypeStruct((ROT_ROWS, B), jnp.float32),
            jax.ShapeDtypeStruct((TRA_ROWS, B), jnp.float32),
            jax.ShapeDtypeStruct((SCA_ROWS, B), jnp.float32),
        ),
        scratch_types=[
            pltpu.VMEM((V,), jnp.float32),
            pltpu.VMEM((B,), jnp.int32),
            pltpu.VMEM((OUT_CHUNK,), jnp.float32),
        ],
        compiler_params=pltpu.CompilerParams(use_tc_tiling_on_sc=True,
                                             needs_layout_passes=False),
    )
    def lane_gather(table_hbm, idx_hbm, rot_hbm, tra_hbm, sca_hbm,
                    col_v, idx_v, out_v):
        wid = lax.axis_index("s") * NC + lax.axis_index("c")
        pltpu.sync_copy(idx_hbm, idx_v)
        for j in range(rows_per_w):
            r = j * NW + wid
            lo, hi = j * NW, j * NW + NW
            targets = []
            if lo < ROT_ROWS and hi > 0:
                targets.append((rot_hbm, 0))
            if lo < ROT_ROWS + TRA_ROWS and hi > ROT_ROWS:
                targets.append((tra_hbm, ROT_ROWS))
            if hi > ROT_ROWS + TRA_ROWS:
                targets.append((sca_hbm, ROT_ROWS + TRA_ROWS))

            @pl.when(r < ROW)
            def _process_row():
                pltpu.sync_copy(table_hbm.at[r], col_v)
                for jc in range(nchunk):
                    def body(i, _):
                        b0 = i * (UNROLL * L)
                        for u in range(UNROLL):
                            o = b0 + u * L
                            g = plsc.load_gather(
                                col_v,
                                [idx_v[pl.ds(jc * OUT_CHUNK + o, L)]])
                            out_v[pl.ds(o, L)] = g
                        return 0
                    lax.fori_loop(0, OUT_CHUNK // (UNROLL * L), body, 0)
                    for out_ref, base in targets:
                        nrows = out_ref.shape[0]
                        if len(targets) == 1:
                            pltpu.sync_copy(
                                out_v,
                                out_ref.at[r - base,
                                           pl.ds(jc * OUT_CHUNK, OUT_CHUNK)])
                        else:
                            @pl.when((r >= base) & (r < base + nrows))
                            def _store():
                                pltpu.sync_copy(
                                    out_v,
                                    out_ref.at[r - base,
                                               pl.ds(jc * OUT_CHUNK,
                                                     OUT_CHUNK)])

    return lane_gather


def kernel(input, weight):
    B = input.shape[0]
    V = weight.shape[0]
    tableT = weight.transpose(2, 1, 0).reshape(ROW, V)
    rot_f, tra_f, sca_f = _make_gather(B, V)(tableT, input)
    rotation = rot_f.reshape(4, N_PARTS, B).transpose(2, 1, 0)
    translation = tra_f.reshape(3, N_PARTS, B).transpose(2, 1, 0)
    scale = sca_f.reshape(3, N_PARTS, B).transpose(2, 1, 0)
    return rotation, translation, scale
